# baseline (speedup 1.0000x reference)
import functools

import jax
import jax.numpy as jnp
from jax import lax
from jax.experimental import pallas as pl
from jax.experimental.pallas import tpu as pltpu
from jax.experimental.pallas import tpu_sc as plsc

B = 4
T = 2048
D = 64
V = 1000000
NC = 2
NS = 16
NW = NC * NS
N = B * T
CHUNK = N // NW
HALF = CHUNK // 2
LANES = 16


def _emb_body(x_hbm, wte_hbm, wtp_hbm, out_hbm,
              idx_v, row_v, off_v, g_v, rows_v, pos_v, sem):
    wid = lax.axis_index("s") * NC + lax.axis_index("c")
    base = wid * CHUNK
    pos_off = lax.rem(base, T)

    pltpu.sync_copy(x_hbm.at[pl.ds(base, CHUNK)], idx_v)

    for g in range(CHUNK // LANES):
        sl = pl.ds(g * LANES, LANES)
        v = idx_v[sl]
        row_v[sl] = v >> 1
        off_v[sl] = (v & 1) << 6

    cp0 = pltpu.async_copy(
        wte_hbm.at[row_v.at[pl.ds(0, HALF)]],
        g_v.at[pl.ds(0, HALF)], sem)
    cp1 = pltpu.async_copy(
        wte_hbm.at[row_v.at[pl.ds(HALF, HALF)]],
        g_v.at[pl.ds(HALF, HALF)], sem)

    pltpu.sync_copy(wtp_hbm.at[pl.ds(pos_off, CHUNK)], pos_v)
    cp0.wait()
    cp1.wait()

    lane_iota = lax.iota(jnp.int32, LANES)

    def add_row(r, carry):
        r16 = jnp.full((LANES,), r, jnp.int32)
        off16 = plsc.load_gather(off_v, [r16])
        for c in range(0, D, LANES):
            col = off16 + (c + lane_iota)
            vals = plsc.load_gather(g_v, [r16, col])
            rows_v[r, pl.ds(c, LANES)] = vals + pos_v[r, pl.ds(c, LANES)]
        return carry

    lax.fori_loop(0, CHUNK, add_row, 0)

    pltpu.sync_copy(rows_v, out_hbm.at[pl.ds(base, CHUNK)])


@jax.jit
def _emb_lookup(x_flat, wte2, wtp):
    mesh = plsc.VectorSubcoreMesh(core_axis_name="c", subcore_axis_name="s")
    return pl.kernel(
        _emb_body,
        out_type=jax.ShapeDtypeStruct((N, D), jnp.float32),
        mesh=mesh,
        scratch_types=[
            pltpu.VMEM((CHUNK,), jnp.int32),
            pltpu.VMEM((CHUNK,), jnp.int32),
            pltpu.VMEM((CHUNK,), jnp.int32),
            pltpu.VMEM((CHUNK, 2 * D), jnp.float32),
            pltpu.VMEM((CHUNK, D), jnp.float32),
            pltpu.VMEM((CHUNK, D), jnp.float32),
            pltpu.SemaphoreType.DMA,
        ],
        compiler_params=pltpu.CompilerParams(needs_layout_passes=False),
    )(x_flat, wte2, wtp)


def kernel(x, wte, wtp):
    out = _emb_lookup(x.reshape(-1), wte.reshape(V // 2, 2 * D), wtp)
    return out.reshape(B, T, D)


# speedup vs baseline: 4.5165x; 4.5165x over previous
import functools

import jax
import jax.numpy as jnp
from jax import lax
from jax.experimental import pallas as pl
from jax.experimental.pallas import tpu as pltpu
from jax.experimental.pallas import tpu_sc as plsc

B = 4
T = 2048
D = 64
V = 1000000
NC = 2
NS = 16
NW = NC * NS
N = B * T
CHUNK = N // NW
LANES = 16
RING = 4


def _emb_body(x_hbm, wt_hbm, wtp_hbm, out_hbm,
              idx_v, colbuf_v, rows_v, pos_v, *sems):
    wid = lax.axis_index("s") * NC + lax.axis_index("c")
    base = wid * CHUNK
    pos_off = lax.rem(base, T)

    pltpu.sync_copy(x_hbm.at[pl.ds(base, CHUNK)], idx_v)
    pltpu.sync_copy(wtp_hbm.at[pl.ds(pos_off, CHUNK)], pos_v)

    lane_iota = lax.iota(jnp.int32, LANES)

    def idx_scalar(i):
        v16 = idx_v[pl.ds((i // LANES) * LANES, LANES)]
        return lax.reduce_max(jnp.where(lane_iota == (i % LANES), v16, 0),
                              axes=(0,))

    def fire(i, r):
        v = idx_scalar(i)
        col = pl.multiple_of((v >> 7) * 128, 128)
        return pltpu.async_copy(
            wt_hbm.at[:, pl.ds(col, 128)], colbuf_v.at[r], sems[r])

    for r in range(RING):
        fire(r, r)

    def do_block(blk, carry):
        for r in range(RING):
            i = blk * RING + r
            pltpu.make_async_copy(
                wt_hbm.at[:, pl.ds(0, 128)], colbuf_v.at[r], sems[r]).wait()
            v = idx_scalar(i)
            lane = jnp.full((LANES,), v & 127, jnp.int32)
            for q in range(0, D, LANES):
                vals = plsc.load_gather(colbuf_v, [
                    jnp.full((LANES,), r, jnp.int32), q + lane_iota, lane])
                rows_v[i, pl.ds(q, LANES)] = vals + pos_v[i, pl.ds(q, LANES)]

            @pl.when(i + RING < CHUNK)
            def _():
                fire(i + RING, r)
        return carry

    lax.fori_loop(0, CHUNK // RING, do_block, 0)

    pltpu.sync_copy(rows_v, out_hbm.at[pl.ds(base, CHUNK)])


@jax.jit
def _emb_lookup(x_flat, wt, wtp):
    mesh = plsc.VectorSubcoreMesh(core_axis_name="c", subcore_axis_name="s")
    return pl.kernel(
        _emb_body,
        out_type=jax.ShapeDtypeStruct((N, D), jnp.float32),
        mesh=mesh,
        scratch_types=[
            pltpu.VMEM((CHUNK,), jnp.int32),
            pltpu.VMEM((RING, D, 128), jnp.float32),
            pltpu.VMEM((CHUNK, D), jnp.float32),
            pltpu.VMEM((CHUNK, D), jnp.float32),
        ] + [pltpu.SemaphoreType.DMA] * RING,
        compiler_params=pltpu.CompilerParams(needs_layout_passes=False),
    )(x_flat, wt, wtp)


def kernel(x, wte, wtp):
    out = _emb_lookup(x.reshape(-1), wte.T, wtp)
    return out.reshape(B, T, D)


# speedup vs baseline: 4.7291x; 1.0471x over previous
import functools

import jax
import jax.numpy as jnp
from jax import lax
from jax.experimental import pallas as pl
from jax.experimental.pallas import tpu as pltpu
from jax.experimental.pallas import tpu_sc as plsc

B = 4
T = 2048
D = 64
V = 1000000
NC = 2
NS = 16
NW = NC * NS
N = B * T
CHUNK = N // NW
LANES = 16
RING = 4


def _emb_body(x_hbm, wt_hbm, wtp_hbm, out_hbm,
              idx_v, colbuf_v, rows_v, pos_v, *sems):
    wid = lax.axis_index("s") * NC + lax.axis_index("c")
    base = wid * CHUNK
    b = base // T
    pos_off = lax.rem(base, T)

    pltpu.sync_copy(x_hbm.at[pl.ds(base, CHUNK)], idx_v)
    pltpu.sync_copy(wtp_hbm.at[:, pl.ds(pos_off, CHUNK)], pos_v)

    lane_iota = lax.iota(jnp.int32, LANES)

    def idx_scalar(i):
        v16 = idx_v[pl.ds((i // LANES) * LANES, LANES)]
        return lax.reduce_max(jnp.where(lane_iota == (i % LANES), v16, 0),
                              axes=(0,))

    def fire(i, r):
        v = idx_scalar(i)
        col = pl.multiple_of((v >> 7) * 128, 128)
        return pltpu.async_copy(
            wt_hbm.at[:, pl.ds(col, 128)], colbuf_v.at[r], sems[r])

    for r in range(RING):
        fire(r, r)

    def do_block(blk, carry):
        for r in range(RING):
            i = blk * RING + r
            pltpu.make_async_copy(
                wt_hbm.at[:, pl.ds(0, 128)], colbuf_v.at[r], sems[r]).wait()
            v = idx_scalar(i)
            i16 = jnp.full((LANES,), i, jnp.int32)
            lane = jnp.full((LANES,), v & 127, jnp.int32)
            for q in range(0, D, LANES):
                vals = plsc.load_gather(colbuf_v, [
                    jnp.full((LANES,), r, jnp.int32), q + lane_iota, lane])
                pvals = plsc.load_gather(pos_v, [q + lane_iota, i16])
                plsc.store_scatter(rows_v, [q + lane_iota, i16], vals + pvals)

            @pl.when(i + RING < CHUNK)
            def _():
                fire(i + RING, r)
        return carry

    lax.fori_loop(0, CHUNK // RING, do_block, 0)

    pltpu.sync_copy(rows_v, out_hbm.at[b, :, pl.ds(pos_off, CHUNK)])


@jax.jit
def _emb_lookup(x_flat, wt, wtp_t):
    mesh = plsc.VectorSubcoreMesh(core_axis_name="c", subcore_axis_name="s")
    return pl.kernel(
        _emb_body,
        out_type=jax.ShapeDtypeStruct((B, D, T), jnp.float32),
        mesh=mesh,
        scratch_types=[
            pltpu.VMEM((CHUNK,), jnp.int32),
            pltpu.VMEM((RING, D, 128), jnp.float32),
            pltpu.VMEM((D, CHUNK), jnp.float32),
            pltpu.VMEM((D, CHUNK), jnp.float32),
        ] + [pltpu.SemaphoreType.DMA] * RING,
        compiler_params=pltpu.CompilerParams(needs_layout_passes=False),
    )(x_flat, wt, wtp_t)


def kernel(x, wte, wtp):
    out_t = _emb_lookup(x.reshape(-1), wte.T, wtp.T)
    return out_t.transpose(0, 2, 1)
